# Initial kernel scaffold; baseline (speedup 1.0000x reference)
import jax
import jax.numpy as jnp
from jax.experimental import pallas as pl


def kernel(x, router_w, expert_weights):
    raise NotImplementedError("write your pallas kernel here")


# speedup vs baseline: 2.2756x; 2.2756x over previous
import functools

import jax
import jax.numpy as jnp
from jax.experimental import pallas as pl
from jax.experimental.pallas import tpu as pltpu

_E = 64


def _moe_body(x_ref, rw_ref, w_ref, out_ref, gate_ref, eid_ref):
    g = pl.program_id(0)

    @pl.when(g == 0)
    def _():
        xt = x_ref[...]
        logits = jax.lax.dot_general(
            xt, rw_ref[...], (((1,), (1,)), ((), ())),
            preferred_element_type=jnp.float32)
        m = jnp.max(logits, axis=1, keepdims=True)
        s = jnp.sum(jnp.exp(logits - m), axis=1, keepdims=True)
        iota = jax.lax.broadcasted_iota(jnp.int32, logits.shape, 1)
        big = jnp.where(logits == m, iota, _E)
        eid_ref[...] = jnp.min(big, axis=1, keepdims=True)
        gate_ref[...] = 1.0 / s

    y = jnp.dot(x_ref[...], w_ref[0], preferred_element_type=jnp.float32)
    coef = jnp.where(eid_ref[...] == g, gate_ref[...], 0.0)
    contrib = coef * y

    @pl.when(g == 0)
    def _():
        out_ref[...] = contrib

    @pl.when(g > 0)
    def _():
        out_ref[...] += contrib


def kernel(x, router_w, expert_weights):
    B, S, H = x.shape
    E, _, D = expert_weights.shape
    T = B * S
    xt = x.reshape(T, H)

    out = pl.pallas_call(
        _moe_body,
        grid=(E,),
        in_specs=[
            pl.BlockSpec((T, H), lambda g: (0, 0)),
            pl.BlockSpec((E, H), lambda g: (0, 0)),
            pl.BlockSpec((1, H, D), lambda g: (g, 0, 0)),
        ],
        out_specs=pl.BlockSpec((T, D), lambda g: (0, 0)),
        out_shape=jax.ShapeDtypeStruct((T, D), jnp.float32),
        scratch_shapes=[
            pltpu.VMEM((T, 1), jnp.float32),
            pltpu.VMEM((T, 1), jnp.int32),
        ],
    )(xt, router_w, expert_weights)
    return out.reshape(B, S, D)
